# baseline (speedup 1.0000x reference)
import functools

import jax
import jax.numpy as jnp
from jax import lax
from jax.experimental import pallas as pl
from jax.experimental.pallas import tpu as pltpu
from jax.experimental.pallas import tpu_sc as plsc

N = 50000
E = 3200000
IN_DIM = 1000
HID = 16

NC, NS = 2, 16
NW = NC * NS
BATCH = 128
OPS = 8
CHUNK = BATCH * OPS

N_PAD = 50176
STRIPE = N_PAD // NS
E_PAD = ((E + NW * CHUNK - 1) // (NW * CHUNK)) * (NW * CHUNK)
ROWS_PER_TILE = E_PAD // NW // BATCH
CHUNKS_PER_TILE = ROWS_PER_TILE // OPS

BR = 6272
BM = 2000

_MESH = plsc.VectorSubcoreMesh(core_axis_name="c", subcore_axis_name="s")
_SC_PARAMS = pltpu.CompilerParams(use_tc_tiling_on_sc=False)


DEGW = 8


@functools.partial(
    pl.kernel,
    out_type=jax.ShapeDtypeStruct((NC * N_PAD, DEGW), jnp.float32),
    mesh=_MESH,
    scratch_types=[
        pltpu.VMEM_SHARED((N_PAD, DEGW), jnp.float32),
        pltpu.VMEM((2, OPS, BATCH), jnp.int32),
        pltpu.VMEM((BATCH, DEGW), jnp.float32),
        pltpu.SemaphoreType.DMA,
        pltpu.SemaphoreType.DMA,
        pltpu.SemaphoreType.DMA,
        pltpu.SemaphoreType.DMA,
    ],
    compiler_params=_SC_PARAMS,
)
def _deg_kernel(dst_hbm, zerosd_hbm, ones_hbm, out_hbm, dacc, dstv, onesv,
                sem0, sem1, ssem0, ssem1):
    sems = (sem0, sem1)
    ssems = (ssem0, ssem1)
    c = lax.axis_index("c")
    s = lax.axis_index("s")
    wid = c * NS + s

    pltpu.sync_copy(ones_hbm, onesv)
    pltpu.sync_copy(zerosd_hbm, dacc.at[pl.ds(s * STRIPE, STRIPE)])
    plsc.subcore_barrier()

    row_base = wid * ROWS_PER_TILE

    def fire_g(k, p):
        rb = row_base + k * OPS
        pltpu.async_copy(dst_hbm.at[pl.ds(rb, OPS)], dstv.at[p], sems[p])

    def wait_g(k, p):
        rb = row_base + k * OPS
        pltpu.make_async_copy(dst_hbm.at[pl.ds(rb, OPS)], dstv.at[p],
                              sems[p]).wait()

    def fire_s(p):
        for j in range(OPS):
            pltpu.async_copy(onesv, dacc.at[dstv.at[p, j]], ssems[p],
                             add=True)

    def wait_s(p):
        for j in range(OPS):
            pltpu.make_async_copy(onesv, dacc.at[dstv.at[p, j]],
                                  ssems[p]).wait()

    def phase(k, p, guarded):
        wait_g(k, p)
        fire_s(p)
        wait_s(1 - p)
        if guarded:
            @pl.when(k + 1 < CHUNKS_PER_TILE)
            def _():
                fire_g(k + 1, 1 - p)
        else:
            fire_g(k + 1, 1 - p)

    fire_g(0, 0)
    wait_g(0, 0)
    fire_s(0)
    fire_g(1, 1)
    wait_g(1, 1)
    fire_s(1)
    wait_s(0)
    fire_g(2, 0)

    def body(i, carry):
        phase(2 * i + 2, 0, False)
        phase(2 * i + 3, 1, True)
        return carry

    lax.fori_loop(0, (CHUNKS_PER_TILE - 2) // 2, body, 0)
    wait_s(1)
    plsc.subcore_barrier()
    pltpu.sync_copy(dacc.at[pl.ds(s * STRIPE, STRIPE)],
                    out_hbm.at[pl.ds(c * N_PAD + s * STRIPE, STRIPE)])


@functools.partial(
    pl.kernel,
    out_type=jax.ShapeDtypeStruct((NC * N_PAD, HID), jnp.float32),
    mesh=_MESH,
    scratch_types=[
        pltpu.VMEM_SHARED((N_PAD, HID), jnp.float32),
        pltpu.VMEM((2, OPS, BATCH), jnp.int32),
        pltpu.VMEM((2, OPS, BATCH), jnp.int32),
        pltpu.VMEM((2 * CHUNK, HID), jnp.float32),
        pltpu.SemaphoreType.DMA,
        pltpu.SemaphoreType.DMA,
        pltpu.SemaphoreType.DMA,
        pltpu.SemaphoreType.DMA,
    ],
    compiler_params=_SC_PARAMS,
)
def _agg_kernel(hp_hbm, src_hbm, dst_hbm, zeros2_hbm, out_hbm,
                acc, srcv, dstv, rows, sem0, sem1, ssem0, ssem1):
    sems = (sem0, sem1)
    ssems = (ssem0, ssem1)
    c = lax.axis_index("c")
    s = lax.axis_index("s")
    wid = c * NS + s

    pltpu.sync_copy(zeros2_hbm, acc.at[pl.ds(s * STRIPE, STRIPE)])
    plsc.subcore_barrier()

    row_base = wid * ROWS_PER_TILE

    def rslice(p, j):
        return rows.at[pl.ds((p * OPS + j) * BATCH, BATCH)]

    def fire_g(k, p):
        rb = row_base + k * OPS
        pltpu.sync_copy(src_hbm.at[pl.ds(rb, OPS)], srcv.at[p])
        pltpu.sync_copy(dst_hbm.at[pl.ds(rb, OPS)], dstv.at[p])
        for j in range(OPS):
            pltpu.async_copy(hp_hbm.at[srcv.at[p, j]], rslice(p, j), sems[p])

    def wait_g(p):
        for j in range(OPS):
            pltpu.make_async_copy(hp_hbm.at[srcv.at[p, j]], rslice(p, j),
                                  sems[p]).wait()

    def fire_s(p):
        for j in range(OPS):
            pltpu.async_copy(rslice(p, j), acc.at[dstv.at[p, j]],
                             ssems[p], add=True)

    def wait_s(p):
        for j in range(OPS):
            pltpu.make_async_copy(rslice(p, j), acc.at[dstv.at[p, j]],
                                  ssems[p]).wait()

    def phase(k, p, guarded):
        wait_g(p)
        fire_s(p)
        wait_s(1 - p)
        if guarded:
            @pl.when(k + 1 < CHUNKS_PER_TILE)
            def _():
                fire_g(k + 1, 1 - p)
        else:
            fire_g(k + 1, 1 - p)

    fire_g(0, 0)
    wait_g(0)
    fire_s(0)
    fire_g(1, 1)
    wait_g(1)
    fire_s(1)
    wait_s(0)
    fire_g(2, 0)

    def body(i, carry):
        phase(2 * i + 2, 0, False)
        phase(2 * i + 3, 1, True)
        return carry

    lax.fori_loop(0, (CHUNKS_PER_TILE - 2) // 2, body, 0)
    wait_s(1)
    plsc.subcore_barrier()
    pltpu.sync_copy(acc.at[pl.ds(s * STRIPE, STRIPE)],
                    out_hbm.at[pl.ds(c * N_PAD + s * STRIPE, STRIPE)])


def _mm_body(x_ref, w_ref, o_ref):
    o_ref[...] = jnp.dot(x_ref[...], w_ref[...],
                         preferred_element_type=jnp.float32)


def _matmul(x, W1):
    return pl.pallas_call(
        _mm_body,
        grid=(N // BM,),
        in_specs=[
            pl.BlockSpec((BM, IN_DIM), lambda i: (i, 0)),
            pl.BlockSpec((IN_DIM, HID), lambda i: (0, 0)),
        ],
        out_specs=pl.BlockSpec((BM, HID), lambda i: (i, 0)),
        out_shape=jax.ShapeDtypeStruct((N, HID), jnp.float32),
    )(x, W1)


def _prep1_body(deg3_ref, h_ref, hp_ref, d16_ref):
    i = pl.program_id(0)
    deg = deg3_ref[0] + deg3_ref[1] + 1.0
    d = 1.0 / jnp.sqrt(deg)
    d_wide = jnp.concatenate([d] * (HID // DEGW), axis=1)
    row = i * BR + lax.broadcasted_iota(jnp.int32, (BR, HID), 0)
    d16 = jnp.where(row < N, d_wide, 0.0)
    d16_ref[...] = d16
    hp_ref[...] = d16 * h_ref[...]


def _prep1(deg3, h_pad):
    return pl.pallas_call(
        _prep1_body,
        grid=(N_PAD // BR,),
        in_specs=[
            pl.BlockSpec((2, BR, DEGW), lambda i: (0, i, 0)),
            pl.BlockSpec((BR, HID), lambda i: (i, 0)),
        ],
        out_specs=[
            pl.BlockSpec((BR, HID), lambda i: (i, 0)),
            pl.BlockSpec((BR, HID), lambda i: (i, 0)),
        ],
        out_shape=[
            jax.ShapeDtypeStruct((N_PAD, HID), jnp.float32),
            jax.ShapeDtypeStruct((N_PAD, HID), jnp.float32),
        ],
    )(deg3, h_pad)


def _prep2_body(a3_ref, hp_ref, d16_ref, b1_ref, ap_ref):
    agg = a3_ref[0] + a3_ref[1] + hp_ref[...]
    t = d16_ref[...] * agg + b1_ref[...]
    ap_ref[...] = d16_ref[...] * jnp.maximum(t, 0.0)


def _prep2(A3, hp, d16, b1row):
    return pl.pallas_call(
        _prep2_body,
        grid=(N_PAD // BR,),
        in_specs=[
            pl.BlockSpec((2, BR, HID), lambda i: (0, i, 0)),
            pl.BlockSpec((BR, HID), lambda i: (i, 0)),
            pl.BlockSpec((BR, HID), lambda i: (i, 0)),
            pl.BlockSpec((1, HID), lambda i: (0, 0)),
        ],
        out_specs=pl.BlockSpec((BR, HID), lambda i: (i, 0)),
        out_shape=jax.ShapeDtypeStruct((N_PAD, HID), jnp.float32),
    )(A3, hp, d16, b1row)


def _final_body(b3_ref, ap_ref, d16_ref, w2_ref, b2_ref, o_ref):
    t = d16_ref[...] * (b3_ref[0] + b3_ref[1] + ap_ref[...])
    o_ref[...] = jnp.dot(t, w2_ref[...],
                         preferred_element_type=jnp.float32) + b2_ref[...]


def _final(B3, ap, d16, W2, b2row):
    return pl.pallas_call(
        _final_body,
        grid=(N_PAD // BR,),
        in_specs=[
            pl.BlockSpec((2, BR, HID), lambda i: (0, i, 0)),
            pl.BlockSpec((BR, HID), lambda i: (i, 0)),
            pl.BlockSpec((BR, HID), lambda i: (i, 0)),
            pl.BlockSpec((HID, 2), lambda i: (0, 0)),
            pl.BlockSpec((1, 2), lambda i: (0, 0)),
        ],
        out_specs=pl.BlockSpec((BR, 2), lambda i: (i, 0)),
        out_shape=jax.ShapeDtypeStruct((N_PAD, 2), jnp.float32),
    )(B3, ap, d16, W2, b2row)


def kernel(x, edge_index, W1, b1, W2, b2):
    src = edge_index[0].astype(jnp.int32)
    dst = edge_index[1].astype(jnp.int32)
    pad = E_PAD - E
    padv = jnp.full((pad,), N, jnp.int32)
    src2d = jnp.concatenate([src, padv]).reshape(E_PAD // BATCH, BATCH)
    dst2d = jnp.concatenate([dst, padv]).reshape(E_PAD // BATCH, BATCH)

    zeros2 = jnp.zeros((STRIPE, HID), jnp.float32)
    zerosd = jnp.zeros((STRIPE, DEGW), jnp.float32)
    onesd = jnp.ones((BATCH, DEGW), jnp.float32)

    deg = _deg_kernel(dst2d, zerosd, onesd)

    h = _matmul(x, W1)
    h_pad = jnp.pad(h, ((0, N_PAD - N), (0, 0)))

    hp, d16 = _prep1(deg.reshape(2, N_PAD, DEGW), h_pad)

    A = _agg_kernel(hp, src2d, dst2d, zeros2)
    ap = _prep2(A.reshape(2, N_PAD, HID), hp, d16, b1.reshape(1, HID))

    B = _agg_kernel(ap, src2d, dst2d, zeros2)
    out = _final(B.reshape(2, N_PAD, HID), ap, d16, W2, b2.reshape(1, 2))
    return out[:N]


# speedup vs baseline: 1.1760x; 1.1760x over previous
import functools

import jax
import jax.numpy as jnp
from jax import lax
from jax.experimental import pallas as pl
from jax.experimental.pallas import tpu as pltpu
from jax.experimental.pallas import tpu_sc as plsc

N = 50000
E = 3200000
IN_DIM = 1000
HID = 16

NC, NS = 2, 16
NW = NC * NS
BATCH = 128
OPS = 8
CHUNK = BATCH * OPS

N_PAD = 50176
STRIPE = N_PAD // NS
E_PAD = ((E + NW * CHUNK - 1) // (NW * CHUNK)) * (NW * CHUNK)
ROWS_PER_TILE = E_PAD // NW // BATCH
CHUNKS_PER_TILE = ROWS_PER_TILE // OPS
assert CHUNKS_PER_TILE % 4 == 2

BR = 6272
BM = 2000

_MESH = plsc.VectorSubcoreMesh(core_axis_name="c", subcore_axis_name="s")
_SC_PARAMS = pltpu.CompilerParams(use_tc_tiling_on_sc=False)


DEGW = 8


@functools.partial(
    pl.kernel,
    out_type=jax.ShapeDtypeStruct((NC * N_PAD, DEGW), jnp.float32),
    mesh=_MESH,
    scratch_types=[
        pltpu.VMEM_SHARED((N_PAD, DEGW), jnp.float32),
        pltpu.VMEM((2, OPS, BATCH), jnp.int32),
        pltpu.VMEM((BATCH, DEGW), jnp.float32),
        pltpu.SemaphoreType.DMA,
        pltpu.SemaphoreType.DMA,
        pltpu.SemaphoreType.DMA,
        pltpu.SemaphoreType.DMA,
    ],
    compiler_params=_SC_PARAMS,
)
def _deg_kernel(dst_hbm, zerosd_hbm, ones_hbm, out_hbm, dacc, dstv, onesv,
                sem0, sem1, ssem0, ssem1):
    sems = (sem0, sem1)
    ssems = (ssem0, ssem1)
    c = lax.axis_index("c")
    s = lax.axis_index("s")
    wid = c * NS + s

    pltpu.sync_copy(ones_hbm, onesv)
    pltpu.sync_copy(zerosd_hbm, dacc.at[pl.ds(s * STRIPE, STRIPE)])
    plsc.subcore_barrier()

    row_base = wid * ROWS_PER_TILE

    def fire_g(k, p):
        rb = row_base + k * OPS
        pltpu.async_copy(dst_hbm.at[pl.ds(rb, OPS)], dstv.at[p], sems[p])

    def wait_g(k, p):
        rb = row_base + k * OPS
        pltpu.make_async_copy(dst_hbm.at[pl.ds(rb, OPS)], dstv.at[p],
                              sems[p]).wait()

    def fire_s(p):
        for j in range(OPS):
            pltpu.async_copy(onesv, dacc.at[dstv.at[p, j]], ssems[p],
                             add=True)

    def wait_s(p):
        for j in range(OPS):
            pltpu.make_async_copy(onesv, dacc.at[dstv.at[p, j]],
                                  ssems[p]).wait()

    def phase(k, p, guarded):
        wait_g(k, p)
        fire_s(p)
        wait_s(1 - p)
        if guarded:
            @pl.when(k + 1 < CHUNKS_PER_TILE)
            def _():
                fire_g(k + 1, 1 - p)
        else:
            fire_g(k + 1, 1 - p)

    fire_g(0, 0)
    wait_g(0, 0)
    fire_s(0)
    fire_g(1, 1)
    wait_g(1, 1)
    fire_s(1)
    wait_s(0)
    fire_g(2, 0)

    def body(i, carry):
        phase(2 * i + 2, 0, False)
        phase(2 * i + 3, 1, True)
        return carry

    lax.fori_loop(0, (CHUNKS_PER_TILE - 2) // 2, body, 0)
    wait_s(1)
    plsc.subcore_barrier()
    pltpu.sync_copy(dacc.at[pl.ds(s * STRIPE, STRIPE)],
                    out_hbm.at[pl.ds(c * N_PAD + s * STRIPE, STRIPE)])


@functools.partial(
    pl.kernel,
    out_type=jax.ShapeDtypeStruct((NC * N_PAD, HID), jnp.float32),
    mesh=_MESH,
    scratch_types=[
        pltpu.VMEM_SHARED((N_PAD, HID), jnp.float32),
        pltpu.VMEM((4, OPS, BATCH), jnp.int32),
        pltpu.VMEM((4, OPS, BATCH), jnp.int32),
        pltpu.VMEM((2 * CHUNK, HID), jnp.float32),
        pltpu.SemaphoreType.DMA,
        pltpu.SemaphoreType.DMA,
        pltpu.SemaphoreType.DMA,
        pltpu.SemaphoreType.DMA,
        pltpu.SemaphoreType.DMA,
        pltpu.SemaphoreType.DMA,
        pltpu.SemaphoreType.DMA,
        pltpu.SemaphoreType.DMA,
    ],
    compiler_params=_SC_PARAMS,
)
def _agg_kernel(hp_hbm, src_hbm, dst_hbm, zeros2_hbm, out_hbm,
                acc, srcv, dstv, rows,
                gsem0, gsem1, ssem0, ssem1, isem0, isem1, isem2, isem3):
    gsems = (gsem0, gsem1)
    ssems = (ssem0, ssem1)
    isems = (isem0, isem1, isem2, isem3)
    c = lax.axis_index("c")
    s = lax.axis_index("s")
    wid = c * NS + s

    pltpu.sync_copy(zeros2_hbm, acc.at[pl.ds(s * STRIPE, STRIPE)])
    plsc.subcore_barrier()

    row_base = wid * ROWS_PER_TILE
    NCH = CHUNKS_PER_TILE

    def rblk(p):
        return rows.at[pl.ds(p * CHUNK, CHUNK)]

    def rslice(p, j):
        return rows.at[pl.ds((p * OPS + j) * BATCH, BATCH)]

    def fire_i(k, q):
        rb = row_base + k * OPS
        pltpu.async_copy(src_hbm.at[pl.ds(rb, OPS)], srcv.at[q], isems[q])
        pltpu.async_copy(dst_hbm.at[pl.ds(rb, OPS)], dstv.at[q], isems[q])

    def wait_i(k, q):
        rb = row_base + k * OPS
        pltpu.make_async_copy(src_hbm.at[pl.ds(rb, OPS)], srcv.at[q],
                              isems[q]).wait()
        pltpu.make_async_copy(dst_hbm.at[pl.ds(rb, OPS)], dstv.at[q],
                              isems[q]).wait()

    def fire_g(p, q):
        for j in range(OPS):
            pltpu.async_copy(hp_hbm.at[srcv.at[q, j]], rslice(p, j), gsems[p])

    def wait_g(p):
        pltpu.make_async_copy(hp_hbm.at[pl.ds(0, CHUNK)], rblk(p),
                              gsems[p]).wait()

    def fire_s(p, q):
        for j in range(OPS):
            pltpu.async_copy(rslice(p, j), acc.at[dstv.at[q, j]],
                             ssems[p], add=True)

    def wait_s(p):
        pltpu.make_async_copy(hp_hbm.at[pl.ds(0, CHUNK)], rblk(p),
                              ssems[p]).wait()

    def phase(k, p, q, first, fire_guarded, prefetch_guarded):
        wait_g(p)
        fire_s(p, q)
        if not first:
            wait_s(1 - p)

        def advance():
            wait_i(k + 1, (q + 1) % 4)
            fire_g(1 - p, (q + 1) % 4)

        if fire_guarded:
            @pl.when(k + 1 < NCH)
            def _():
                advance()
        else:
            advance()

        def prefetch():
            fire_i(k + 3, (q + 3) % 4)

        if prefetch_guarded:
            @pl.when(k + 3 < NCH)
            def _():
                prefetch()
        elif not first or k >= 1:
            prefetch()

    fire_i(0, 0)
    fire_i(1, 1)
    fire_i(2, 2)
    wait_i(0, 0)
    fire_g(0, 0)
    phase(0, 0, 0, True, False, False)
    fire_i(3, 3)
    phase(1, 1, 1, False, False, False)

    def body(i, carry):
        k = 4 * i + 2
        phase(k + 0, 0, 2, False, False, False)
        phase(k + 1, 1, 3, False, False, False)
        phase(k + 2, 0, 0, False, False, False)
        phase(k + 3, 1, 1, False, False, False)
        return carry

    lax.fori_loop(0, (NCH - 6) // 4, body, 0)
    base = NCH - 4
    phase(base + 0, 0, 2, False, False, True)
    phase(base + 1, 1, 3, False, False, True)
    phase(base + 2, 0, 0, False, False, True)
    phase(base + 3, 1, 1, False, True, True)
    wait_s((NCH - 1) % 2)
    plsc.subcore_barrier()
    pltpu.sync_copy(acc.at[pl.ds(s * STRIPE, STRIPE)],
                    out_hbm.at[pl.ds(c * N_PAD + s * STRIPE, STRIPE)])


def _mm_body(x_ref, w_ref, o_ref):
    o_ref[...] = jnp.dot(x_ref[...], w_ref[...],
                         preferred_element_type=jnp.float32)


def _matmul(x, W1):
    return pl.pallas_call(
        _mm_body,
        grid=(N // BM,),
        in_specs=[
            pl.BlockSpec((BM, IN_DIM), lambda i: (i, 0)),
            pl.BlockSpec((IN_DIM, HID), lambda i: (0, 0)),
        ],
        out_specs=pl.BlockSpec((BM, HID), lambda i: (i, 0)),
        out_shape=jax.ShapeDtypeStruct((N, HID), jnp.float32),
    )(x, W1)


def _prep1_body(deg3_ref, h_ref, hp_ref, d16_ref):
    i = pl.program_id(0)
    deg = deg3_ref[0] + deg3_ref[1] + 1.0
    d = 1.0 / jnp.sqrt(deg)
    d_wide = jnp.concatenate([d] * (HID // DEGW), axis=1)
    row = i * BR + lax.broadcasted_iota(jnp.int32, (BR, HID), 0)
    d16 = jnp.where(row < N, d_wide, 0.0)
    d16_ref[...] = d16
    hp_ref[...] = d16 * h_ref[...]


def _prep1(deg3, h_pad):
    return pl.pallas_call(
        _prep1_body,
        grid=(N_PAD // BR,),
        in_specs=[
            pl.BlockSpec((2, BR, DEGW), lambda i: (0, i, 0)),
            pl.BlockSpec((BR, HID), lambda i: (i, 0)),
        ],
        out_specs=[
            pl.BlockSpec((BR, HID), lambda i: (i, 0)),
            pl.BlockSpec((BR, HID), lambda i: (i, 0)),
        ],
        out_shape=[
            jax.ShapeDtypeStruct((N_PAD, HID), jnp.float32),
            jax.ShapeDtypeStruct((N_PAD, HID), jnp.float32),
        ],
    )(deg3, h_pad)


def _prep2_body(a3_ref, hp_ref, d16_ref, b1_ref, ap_ref):
    agg = a3_ref[0] + a3_ref[1] + hp_ref[...]
    t = d16_ref[...] * agg + b1_ref[...]
    ap_ref[...] = d16_ref[...] * jnp.maximum(t, 0.0)


def _prep2(A3, hp, d16, b1row):
    return pl.pallas_call(
        _prep2_body,
        grid=(N_PAD // BR,),
        in_specs=[
            pl.BlockSpec((2, BR, HID), lambda i: (0, i, 0)),
            pl.BlockSpec((BR, HID), lambda i: (i, 0)),
            pl.BlockSpec((BR, HID), lambda i: (i, 0)),
            pl.BlockSpec((1, HID), lambda i: (0, 0)),
        ],
        out_specs=pl.BlockSpec((BR, HID), lambda i: (i, 0)),
        out_shape=jax.ShapeDtypeStruct((N_PAD, HID), jnp.float32),
    )(A3, hp, d16, b1row)


def _final_body(b3_ref, ap_ref, d16_ref, w2_ref, b2_ref, o_ref):
    t = d16_ref[...] * (b3_ref[0] + b3_ref[1] + ap_ref[...])
    o_ref[...] = jnp.dot(t, w2_ref[...],
                         preferred_element_type=jnp.float32) + b2_ref[...]


def _final(B3, ap, d16, W2, b2row):
    return pl.pallas_call(
        _final_body,
        grid=(N_PAD // BR,),
        in_specs=[
            pl.BlockSpec((2, BR, HID), lambda i: (0, i, 0)),
            pl.BlockSpec((BR, HID), lambda i: (i, 0)),
            pl.BlockSpec((BR, HID), lambda i: (i, 0)),
            pl.BlockSpec((HID, 2), lambda i: (0, 0)),
            pl.BlockSpec((1, 2), lambda i: (0, 0)),
        ],
        out_specs=pl.BlockSpec((BR, 2), lambda i: (i, 0)),
        out_shape=jax.ShapeDtypeStruct((N_PAD, 2), jnp.float32),
    )(B3, ap, d16, W2, b2row)


def kernel(x, edge_index, W1, b1, W2, b2):
    src = edge_index[0].astype(jnp.int32)
    dst = edge_index[1].astype(jnp.int32)
    pad = E_PAD - E
    padv = jnp.full((pad,), N, jnp.int32)
    src2d = jnp.concatenate([src, padv]).reshape(E_PAD // BATCH, BATCH)
    dst2d = jnp.concatenate([dst, padv]).reshape(E_PAD // BATCH, BATCH)

    zeros2 = jnp.zeros((STRIPE, HID), jnp.float32)
    zerosd = jnp.zeros((STRIPE, DEGW), jnp.float32)
    onesd = jnp.ones((BATCH, DEGW), jnp.float32)

    deg = _deg_kernel(dst2d, zerosd, onesd)

    h = _matmul(x, W1)
    h_pad = jnp.pad(h, ((0, N_PAD - N), (0, 0)))

    hp, d16 = _prep1(deg.reshape(2, N_PAD, DEGW), h_pad)

    A = _agg_kernel(hp, src2d, dst2d, zeros2)
    ap = _prep2(A.reshape(2, N_PAD, HID), hp, d16, b1.reshape(1, HID))

    B = _agg_kernel(ap, src2d, dst2d, zeros2)
    out = _final(B.reshape(2, N_PAD, HID), ap, d16, W2, b2.reshape(1, 2))
    return out[:N]
